# baseline (speedup 1.0000x reference)
import functools

import jax
import jax.numpy as jnp
from jax import lax
from jax.experimental import pallas as pl
from jax.experimental.pallas import tpu as pltpu
from jax.experimental.pallas import tpu_sc as plsc

_B = 16
_S = 2048
_NHMM = 30
_HID = 256
_CH = 128
_S_PAD = _S + 8
_NC = 2
_NS = 16
_HALF = _S // 2
_CHUNK = 128
_NCHUNK = _HALF // _CHUNK


def _prep_body(xr_ref, seq_ref, ss_ref, feat_ref, mask_ref):
    xr = xr_ref[0]
    ch0 = xr[0:1, :]
    rest = xr[1:21, :]
    mx = jnp.max(rest, axis=0, keepdims=True)
    mask_ref[0] = (mx > ch0).astype(jnp.int32)
    rows = jnp.concatenate(
        [rest, seq_ref[...], ss_ref[...],
         jnp.zeros((64 - 53, _S), jnp.float32)], axis=0)
    feat_ref[:, 0:64] = rows.T
    feat_ref[:, 64:128] = jnp.zeros((_S, 64), jnp.float32)


def _prep(xr, seq_hmm, ss_hmm):
    return pl.pallas_call(
        _prep_body,
        grid=(_B,),
        in_specs=[
            pl.BlockSpec((1, 21, _S), lambda b: (b, 0, 0)),
            pl.BlockSpec((_NHMM, _S), lambda b: (0, 0)),
            pl.BlockSpec((3, _S), lambda b: (0, 0)),
        ],
        out_specs=[
            pl.BlockSpec((_S, _CH), lambda b: (b, 0)),
            pl.BlockSpec((1, 1, _S), lambda b: (b, 0, 0)),
        ],
        out_shape=[
            jax.ShapeDtypeStruct((_B * _S, _CH), jnp.float32),
            jax.ShapeDtypeStruct((_B, 1, _S), jnp.int32),
        ],
    )(xr, seq_hmm, ss_hmm)


def _compact_body(mask_hbm, feats_hbm, out_hbm, mask_v, dest_v, rows_v, sem):
    wid = lax.axis_index("c") * _NS + lax.axis_index("s")
    b = wid // 2
    half = wid % 2

    pltpu.sync_copy(mask_hbm.at[b, pl.ds(half * _HALF, _HALF)],
                    mask_v.at[pl.ds(0, _HALF)])
    pltpu.sync_copy(mask_hbm.at[b, pl.ds(0, _HALF)],
                    mask_v.at[pl.ds(_HALF, _HALF)])

    def _count(i, acc):
        return acc + mask_v[pl.ds(_HALF + i * 16, 16)]
    accv = lax.fori_loop(0, _HALF // 16, _count, jnp.zeros((16,), jnp.int32))
    base0 = half * jnp.sum(accv)

    dump = b * _S_PAD + _S

    def _dest(i, base):
        m = mask_v[pl.ds(i * 16, 16)]
        c = plsc.cumsum(m)
        d = jnp.where(m != 0, b * _S_PAD + base + c - 1, dump)
        dest_v[i // 8, pl.ds((i % 8) * 16, 16)] = d
        return base + jnp.max(c)
    lax.fori_loop(0, _HALF // 16, _dest, base0)

    src0 = b * _S + half * _HALF

    def _chunk(j, carry):
        pltpu.sync_copy(feats_hbm.at[pl.ds(src0 + j * _CHUNK, _CHUNK)], rows_v)
        pltpu.async_copy(rows_v, out_hbm.at[dest_v.at[j]], sem).wait()
        return carry
    lax.fori_loop(0, _NCHUNK, _chunk, 0)


@functools.lru_cache(maxsize=1)
def _compact_call():
    return pl.kernel(
        _compact_body,
        out_type=jax.ShapeDtypeStruct((_B * _S_PAD, _CH), jnp.float32),
        mesh=plsc.VectorSubcoreMesh(core_axis_name="c", subcore_axis_name="s",
                                    num_cores=_NC, num_subcores=_NS),
        scratch_types=[
            pltpu.VMEM((_S,), jnp.int32),
            pltpu.VMEM((_NCHUNK, _CHUNK), jnp.int32),
            pltpu.VMEM((_CHUNK, _CH), jnp.float32),
            pltpu.SemaphoreType.DMA,
        ],
        compiler_params=pltpu.CompilerParams(needs_layout_passes=False,
                                             use_tc_tiling_on_sc=False),
    )


def _compact(mask2, feats2):
    return _compact_call()(mask2, feats2)


def _shift(a, d):
    if d == 0:
        return a
    z = jnp.zeros((abs(d), a.shape[1]), a.dtype)
    if d > 0:
        return jnp.concatenate([a[d:], z], axis=0)
    return jnp.concatenate([z, a[:d]], axis=0)


def _model_body(comp_ref, mask_ref, w1_ref, b1_ref, w2_ref, b2_ref, out_ref):
    comp = comp_ref[0:_S, :]
    m = mask_ref[0, 0, :]
    ls = jnp.sum(m)
    pos = lax.broadcasted_iota(jnp.int32, (_S, 1), 0)
    valid = pos < ls
    zf = jnp.where(valid, comp, 0.0)
    w3 = zf[:, 50:53]

    u = jnp.dot(zf, w1_ref[...], preferred_element_type=jnp.float32)
    h = _shift(u[:, 0:_HID], -2)
    for k in range(1, 5):
        h = h + _shift(u[:, _HID * k:_HID * (k + 1)], k - 2)
    h = jnp.maximum(h + b1_ref[...], 0.0)
    y = jnp.dot(h, w2_ref[...], preferred_element_type=jnp.float32)

    logits = _shift(y[:, 0:3], -2)
    for k in range(1, 5):
        logits = logits + _shift(y[:, 3 * k:3 * k + 3], k - 2)
    logits = logits + b2_ref[0, :3][None, :]

    e = jnp.exp(logits)
    den = jnp.sum(e, axis=1, keepdims=True)
    num = jnp.sum(w3 * e, axis=1, keepdims=True)
    contrib = jnp.sum(num / den)
    a = jnp.log(contrib / ls.astype(jnp.float32))
    out_ref[0, 0, :] = jnp.full((128,), a, jnp.float32)


def _model(comp2, mask3, w1all, b1r, w2all, b2r):
    return pl.pallas_call(
        _model_body,
        grid=(_B,),
        in_specs=[
            pl.BlockSpec((_S_PAD, _CH), lambda b: (b, 0)),
            pl.BlockSpec((1, 1, _S), lambda b: (b, 0, 0)),
            pl.BlockSpec((_CH, 5 * _HID), lambda b: (0, 0)),
            pl.BlockSpec((1, _HID), lambda b: (0, 0)),
            pl.BlockSpec((_HID, 128), lambda b: (0, 0)),
            pl.BlockSpec((1, 128), lambda b: (0, 0)),
        ],
        out_specs=pl.BlockSpec((1, 1, 128), lambda b: (b, 0, 0)),
        out_shape=jax.ShapeDtypeStruct((_B, 1, 128), jnp.float32),
    )(comp2, mask3, w1all, b1r, w2all, b2r)


def kernel(x, seq_hmm, ss_hmm, W1, b1, W2, b2):
    xr = x.reshape(_B, 21, _S)

    w1t = jnp.transpose(W1, (1, 2, 0)).reshape(50, 5 * _HID)
    w1all = jnp.zeros((_CH, 5 * _HID), jnp.float32).at[:50].set(w1t)
    b1r = b1[None, :]
    w2t = jnp.transpose(W2, (1, 2, 0)).reshape(_HID, 15)
    w2all = jnp.zeros((_HID, 128), jnp.float32).at[:, :15].set(w2t)
    b2r = jnp.zeros((1, 128), jnp.float32).at[0, :3].set(b2)

    feats, mask3 = _prep(xr, seq_hmm, ss_hmm)
    comp = _compact(mask3.reshape(_B, _S), feats)
    out = _model(comp, mask3, w1all, b1r, w2all, b2r)
    return out[:, 0, 0]


# speedup vs baseline: 2.2655x; 2.2655x over previous
import functools

import jax
import jax.numpy as jnp
from jax import lax
from jax.experimental import pallas as pl
from jax.experimental.pallas import tpu as pltpu
from jax.experimental.pallas import tpu_sc as plsc

_B = 16
_S = 2048
_NHMM = 30
_HID = 256
_CH = 128
_S_PAD = _S + 8
_NC = 2
_NS = 16
_HALF = _S // 2
_CHUNK = 128
_NCHUNK = _HALF // _CHUNK


def _prep_body(xr_ref, seq_ref, ss_ref, feat_ref, mask_ref):
    xr = xr_ref[0]
    ch0 = xr[0:1, :]
    rest = xr[1:21, :]
    mx = jnp.max(rest, axis=0, keepdims=True)
    mask_ref[0] = (mx > ch0).astype(jnp.int32)
    rows = jnp.concatenate(
        [rest, seq_ref[...], ss_ref[...],
         jnp.zeros((64 - 53, _S), jnp.float32)], axis=0)
    feat_ref[:, 0:64] = rows.T
    feat_ref[:, 64:128] = jnp.zeros((_S, 64), jnp.float32)


def _prep(xr, seq_hmm, ss_hmm):
    return pl.pallas_call(
        _prep_body,
        grid=(_B,),
        in_specs=[
            pl.BlockSpec((1, 21, _S), lambda b: (b, 0, 0)),
            pl.BlockSpec((_NHMM, _S), lambda b: (0, 0)),
            pl.BlockSpec((3, _S), lambda b: (0, 0)),
        ],
        out_specs=[
            pl.BlockSpec((_S, _CH), lambda b: (b, 0)),
            pl.BlockSpec((1, 1, _S), lambda b: (b, 0, 0)),
        ],
        out_shape=[
            jax.ShapeDtypeStruct((_B * _S, _CH), jnp.float32),
            jax.ShapeDtypeStruct((_B, 1, _S), jnp.int32),
        ],
    )(xr, seq_hmm, ss_hmm)


def _compact_body(mask_hbm, feats_hbm, out_hbm, mask_v, dest_v, rows_v, sem):
    wid = lax.axis_index("c") * _NS + lax.axis_index("s")
    b = wid // 2
    half = wid % 2

    pltpu.sync_copy(mask_hbm.at[b, pl.ds(half * _HALF, _HALF)],
                    mask_v.at[pl.ds(0, _HALF)])
    pltpu.sync_copy(mask_hbm.at[b, pl.ds(0, _HALF)],
                    mask_v.at[pl.ds(_HALF, _HALF)])

    def _count(i, acc):
        return acc + mask_v[pl.ds(_HALF + i * 16, 16)]
    accv = lax.fori_loop(0, _HALF // 16, _count, jnp.zeros((16,), jnp.int32))
    base0 = half * jnp.sum(accv)

    dump = b * _S_PAD + _S

    def _dest(i, base):
        m = mask_v[pl.ds(i * 16, 16)]
        c = plsc.cumsum(m)
        d = jnp.where(m != 0, b * _S_PAD + base + c - 1, dump)
        dest_v[i // 8, pl.ds((i % 8) * 16, 16)] = d
        return base + jnp.max(c)
    lax.fori_loop(0, _HALF // 16, _dest, base0)

    src0 = b * _S + half * _HALF

    def _chunk(j, carry):
        pltpu.sync_copy(feats_hbm.at[pl.ds(src0 + j * _CHUNK, _CHUNK)], rows_v)
        pltpu.async_copy(rows_v, out_hbm.at[dest_v.at[j]], sem).wait()
        return carry
    lax.fori_loop(0, _NCHUNK, _chunk, 0)


@functools.lru_cache(maxsize=1)
def _compact_call():
    return pl.kernel(
        _compact_body,
        out_type=jax.ShapeDtypeStruct((_B * _S_PAD, _CH), jnp.float32),
        mesh=plsc.VectorSubcoreMesh(core_axis_name="c", subcore_axis_name="s",
                                    num_cores=_NC, num_subcores=_NS),
        scratch_types=[
            pltpu.VMEM((_S,), jnp.int32),
            pltpu.VMEM((_NCHUNK, _CHUNK), jnp.int32),
            pltpu.VMEM((_CHUNK, _CH), jnp.float32),
            pltpu.SemaphoreType.DMA,
        ],
        compiler_params=pltpu.CompilerParams(needs_layout_passes=False,
                                             use_tc_tiling_on_sc=False),
    )


def _compact(mask2, feats2):
    return _compact_call()(mask2, feats2)


def _shift(a, d):
    if d == 0:
        return a
    z = jnp.zeros((abs(d), a.shape[1]), a.dtype)
    if d > 0:
        return jnp.concatenate([a[d:], z], axis=0)
    return jnp.concatenate([z, a[:d]], axis=0)


def _shift_lane(a, d):
    if d == 0:
        return a
    z = jnp.zeros((a.shape[0], abs(d)), a.dtype)
    if d > 0:
        return jnp.concatenate([a[:, d:], z], axis=1)
    return jnp.concatenate([z, a[:, :d]], axis=1)


def _model_body(comp_ref, mask_ref, w1_ref, b1_ref, w2_ref, b2_ref, out_ref):
    comp = comp_ref[0:_S, 0:64]
    m = mask_ref[0, 0, :]
    ls = jnp.sum(m)
    pos = lax.broadcasted_iota(jnp.int32, (_S, 1), 0)
    valid = pos < ls
    zf = jnp.where(valid, comp, 0.0)

    x5 = jnp.concatenate([_shift(zf, d) for d in (-2, -1, 0, 1, 2)], axis=1)
    h = jnp.dot(x5, w1_ref[...], preferred_element_type=jnp.float32)
    h = jnp.maximum(h + b1_ref[...], 0.0)

    yt = lax.dot_general(w2_ref[...], h, (((1,), (1,)), ((), ())),
                         preferred_element_type=jnp.float32)
    lt = _shift_lane(yt[0:3, :], -2)
    for k in range(1, 5):
        lt = lt + _shift_lane(yt[3 * k:3 * k + 3, :], k - 2)
    lt = lt + b2_ref[0, 0:3][:, None]

    e = jnp.exp(lt)
    den = e[0:1, :] + e[1:2, :] + e[2:3, :]
    w3t = zf[:, 48:56].T
    num = (w3t[2:3, :] * e[0:1, :] + w3t[3:4, :] * e[1:2, :]
           + w3t[4:5, :] * e[2:3, :])
    contrib = jnp.sum(num / den)
    a = jnp.log(contrib / ls.astype(jnp.float32))
    out_ref[0, 0, :] = jnp.full((128,), a, jnp.float32)


def _model(comp2, mask3, w1all, b1r, w2t16, b2r):
    return pl.pallas_call(
        _model_body,
        grid=(_B,),
        in_specs=[
            pl.BlockSpec((_S_PAD, _CH), lambda b: (b, 0)),
            pl.BlockSpec((1, 1, _S), lambda b: (b, 0, 0)),
            pl.BlockSpec((320, _HID), lambda b: (0, 0)),
            pl.BlockSpec((1, _HID), lambda b: (0, 0)),
            pl.BlockSpec((16, _HID), lambda b: (0, 0)),
            pl.BlockSpec((1, 128), lambda b: (0, 0)),
        ],
        out_specs=pl.BlockSpec((1, 1, 128), lambda b: (b, 0, 0)),
        out_shape=jax.ShapeDtypeStruct((_B, 1, 128), jnp.float32),
    )(comp2, mask3, w1all, b1r, w2t16, b2r)


def kernel(x, seq_hmm, ss_hmm, W1, b1, W2, b2):
    xr = x.reshape(_B, 21, _S)

    w1t = jnp.transpose(W1, (2, 1, 0))
    w1all = (jnp.zeros((5, 64, _HID), jnp.float32)
             .at[:, :50].set(w1t).reshape(320, _HID))
    b1r = b1[None, :]
    w2t = jnp.transpose(W2, (1, 2, 0)).reshape(_HID, 15)
    w2t16 = jnp.zeros((16, _HID), jnp.float32).at[:15].set(w2t.T)
    b2r = jnp.zeros((1, 128), jnp.float32).at[0, :3].set(b2)

    feats, mask3 = _prep(xr, seq_hmm, ss_hmm)
    comp = _compact(mask3.reshape(_B, _S), feats)
    out = _model(comp, mask3, w1all, b1r, w2t16, b2r)
    return out[:, 0, 0]


# speedup vs baseline: 2.4098x; 1.0637x over previous
import functools

import jax
import jax.numpy as jnp
from jax import lax
from jax.experimental import pallas as pl
from jax.experimental.pallas import tpu as pltpu
from jax.experimental.pallas import tpu_sc as plsc

_B = 16
_S = 2048
_NHMM = 30
_HID = 256
_CH = 128
_S_PAD = _S + 8
_NC = 2
_NS = 16
_HALF = _S // 2
_CHUNK = 128
_NCHUNK = _HALF // _CHUNK


def _prep_body(xr_ref, seq_ref, ss_ref, feat_ref, mask_ref):
    xr = xr_ref[0]
    ch0 = xr[0:1, :]
    rest = xr[1:21, :]
    mx = jnp.max(rest, axis=0, keepdims=True)
    mask_ref[0] = (mx > ch0).astype(jnp.int32)
    rows = jnp.concatenate(
        [rest, seq_ref[...], ss_ref[...],
         jnp.zeros((64 - 53, _S), jnp.float32)], axis=0)
    feat_ref[:, 0:64] = rows.T
    feat_ref[:, 64:128] = jnp.zeros((_S, 64), jnp.float32)


def _prep(xr, seq_hmm, ss_hmm):
    return pl.pallas_call(
        _prep_body,
        grid=(_B,),
        in_specs=[
            pl.BlockSpec((1, 21, _S), lambda b: (b, 0, 0)),
            pl.BlockSpec((_NHMM, _S), lambda b: (0, 0)),
            pl.BlockSpec((3, _S), lambda b: (0, 0)),
        ],
        out_specs=[
            pl.BlockSpec((_S, _CH), lambda b: (b, 0)),
            pl.BlockSpec((1, 1, _S), lambda b: (b, 0, 0)),
        ],
        out_shape=[
            jax.ShapeDtypeStruct((_B * _S, _CH), jnp.float32),
            jax.ShapeDtypeStruct((_B, 1, _S), jnp.int32),
        ],
    )(xr, seq_hmm, ss_hmm)


_NBUF = 4


def _compact_body(mask_hbm, feats_hbm, out_hbm, mask_v, dest_v, rows_v,
                  *sems):
    wid = lax.axis_index("c") * _NS + lax.axis_index("s")
    b = wid // 2
    half = wid % 2
    gsems = sems[:_NBUF]
    ssems = sems[_NBUF:]

    src0 = b * _S + half * _HALF

    def _gather(j):
        return pltpu.async_copy(
            feats_hbm.at[pl.ds(src0 + j * _CHUNK, _CHUNK)],
            rows_v.at[j % _NBUF], gsems[j % _NBUF])

    descs_g = [None] * _NCHUNK
    for j in range(_NBUF):
        descs_g[j] = _gather(j)

    m_own = pltpu.async_copy(mask_hbm.at[b, pl.ds(half * _HALF, _HALF)],
                             mask_v.at[pl.ds(0, _HALF)], ssems[0])
    m_low = pltpu.async_copy(mask_hbm.at[b, pl.ds(0, _HALF)],
                             mask_v.at[pl.ds(_HALF, _HALF)], ssems[1])
    m_own.wait()
    m_low.wait()

    def _count(i, acc):
        return acc + mask_v[pl.ds(_HALF + i * 16, 16)]
    accv = lax.fori_loop(0, _HALF // 16, _count, jnp.zeros((16,), jnp.int32))
    base0 = half * jnp.sum(accv)

    dump = b * _S_PAD + _S

    def _dest(i, base):
        ms = [mask_v[pl.ds((4 * i + q) * 16, 16)] for q in range(4)]
        cs = [plsc.cumsum(mq) for mq in ms]
        ts = [jnp.max(cq) for cq in cs]
        for q in range(4):
            d = jnp.where(ms[q] != 0, b * _S_PAD + base + cs[q] - 1, dump)
            dest_v[(4 * i + q) // 8, pl.ds(((4 * i + q) % 8) * 16, 16)] = d
            base = base + ts[q]
        return base
    lax.fori_loop(0, _HALF // 64, _dest, base0)

    descs_s = [None] * _NCHUNK
    for j in range(_NCHUNK):
        descs_g[j].wait()
        if j + 1 < _NCHUNK and descs_g[j + 1] is None:
            descs_s[j + 1 - _NBUF].wait()
            descs_g[j + 1] = _gather(j + 1)
        descs_s[j] = pltpu.async_copy(rows_v.at[j % _NBUF],
                                      out_hbm.at[dest_v.at[j]],
                                      ssems[j % _NBUF])
    for j in range(_NCHUNK - _NBUF, _NCHUNK):
        descs_s[j].wait()


@functools.lru_cache(maxsize=1)
def _compact_call():
    return pl.kernel(
        _compact_body,
        out_type=jax.ShapeDtypeStruct((_B * _S_PAD, _CH), jnp.float32),
        mesh=plsc.VectorSubcoreMesh(core_axis_name="c", subcore_axis_name="s",
                                    num_cores=_NC, num_subcores=_NS),
        scratch_types=[
            pltpu.VMEM((_S,), jnp.int32),
            pltpu.VMEM((_NCHUNK, _CHUNK), jnp.int32),
            pltpu.VMEM((_NBUF, _CHUNK, _CH), jnp.float32),
        ] + [pltpu.SemaphoreType.DMA] * (2 * _NBUF),
        compiler_params=pltpu.CompilerParams(needs_layout_passes=False,
                                             use_tc_tiling_on_sc=False),
    )


def _compact(mask2, feats2):
    return _compact_call()(mask2, feats2)


def _shift(a, d):
    if d == 0:
        return a
    z = jnp.zeros((abs(d), a.shape[1]), a.dtype)
    if d > 0:
        return jnp.concatenate([a[d:], z], axis=0)
    return jnp.concatenate([z, a[:d]], axis=0)


def _shift_lane(a, d):
    if d == 0:
        return a
    z = jnp.zeros((a.shape[0], abs(d)), a.dtype)
    if d > 0:
        return jnp.concatenate([a[:, d:], z], axis=1)
    return jnp.concatenate([z, a[:, :d]], axis=1)


def _model_body(comp_ref, mask_ref, w1_ref, b1_ref, w2_ref, b2_ref, out_ref):
    comp = comp_ref[0:_S, 0:64]
    m = mask_ref[0, 0, :]
    ls = jnp.sum(m)
    pos = lax.broadcasted_iota(jnp.int32, (_S, 1), 0)
    valid = pos < ls
    zf = jnp.where(valid, comp, 0.0)

    zb = zf.astype(jnp.bfloat16)
    x5 = jnp.concatenate([_shift(zb, d) for d in (-2, -1, 0, 1, 2)], axis=1)
    h = jnp.dot(x5, w1_ref[...].astype(jnp.bfloat16),
                preferred_element_type=jnp.float32)
    h = jnp.maximum(h + b1_ref[...], 0.0)

    yt = lax.dot_general(w2_ref[...], h, (((1,), (1,)), ((), ())),
                         preferred_element_type=jnp.float32)
    lt = _shift_lane(yt[0:3, :], -2)
    for k in range(1, 5):
        lt = lt + _shift_lane(yt[3 * k:3 * k + 3, :], k - 2)
    lt = lt + b2_ref[0, 0:3][:, None]

    e = jnp.exp(lt)
    den = e[0:1, :] + e[1:2, :] + e[2:3, :]
    w3t = zf[:, 48:56].T
    num = (w3t[2:3, :] * e[0:1, :] + w3t[3:4, :] * e[1:2, :]
           + w3t[4:5, :] * e[2:3, :])
    contrib = jnp.sum(num / den)
    a = jnp.log(contrib / ls.astype(jnp.float32))
    out_ref[0, 0, :] = jnp.full((128,), a, jnp.float32)


def _model(comp2, mask3, w1all, b1r, w2t16, b2r):
    return pl.pallas_call(
        _model_body,
        grid=(_B,),
        in_specs=[
            pl.BlockSpec((_S_PAD, _CH), lambda b: (b, 0)),
            pl.BlockSpec((1, 1, _S), lambda b: (b, 0, 0)),
            pl.BlockSpec((320, _HID), lambda b: (0, 0)),
            pl.BlockSpec((1, _HID), lambda b: (0, 0)),
            pl.BlockSpec((16, _HID), lambda b: (0, 0)),
            pl.BlockSpec((1, 128), lambda b: (0, 0)),
        ],
        out_specs=pl.BlockSpec((1, 1, 128), lambda b: (b, 0, 0)),
        out_shape=jax.ShapeDtypeStruct((_B, 1, 128), jnp.float32),
    )(comp2, mask3, w1all, b1r, w2t16, b2r)


def kernel(x, seq_hmm, ss_hmm, W1, b1, W2, b2):
    xr = x.reshape(_B, 21, _S)

    w1t = jnp.transpose(W1, (2, 1, 0))
    w1all = jnp.pad(w1t, ((0, 0), (0, 14), (0, 0))).reshape(320, _HID)
    b1r = b1[None, :]
    w2t = jnp.transpose(W2, (2, 0, 1)).reshape(15, _HID)
    w2t16 = jnp.pad(w2t, ((0, 1), (0, 0)))
    b2r = jnp.pad(b2, (0, 125))[None, :]

    feats, mask3 = _prep(xr, seq_hmm, ss_hmm)
    comp = _compact(mask3.reshape(_B, _S), feats)
    out = _model(comp, mask3, w1all, b1r, w2t16, b2r)
    return out[:, 0, 0]
